# baseline (speedup 1.0000x reference)
import jax
import jax.numpy as jnp
from jax.experimental import pallas as pl
from jax.experimental.pallas import tpu as pltpu

B, D, T = 16, 512, 2048
N_CB, K, CD = 9, 1024, 8

TT = 1024
GT = T // TT
CODES_PAD = 16
KQ = K // 128
CDA = 2 * CD

_PREC = None
_PREC_GATHER = jax.lax.Precision.HIGHEST


def _dot(a, b, dims, precision):
    return jax.lax.dot_general(a, b, (dims, ((), ())),
                               preferred_element_type=jnp.float32,
                               precision=precision)


def _prep_kernel(in_v_ref, in_g_ref, out_v_ref, out_g_ref, cb_ref,
                 w_in_ref, w_out_ref, cbn_ref):
    iv = in_v_ref[...]
    w_in_ref[...] = in_g_ref[...][:, :, None] * iv / jnp.sqrt(
        jnp.sum(iv * iv, axis=2, keepdims=True))
    ov = out_v_ref[...]
    w_out_ref[...] = out_g_ref[...][:, :, None] * ov / jnp.sqrt(
        jnp.sum(ov * ov, axis=2, keepdims=True))
    cb = cb_ref[...]
    cbn = cb / jnp.maximum(
        jnp.sqrt(jnp.sum(cb * cb, axis=2, keepdims=True)), 1e-12)
    ch = -0.5 * jnp.sum(cbn * cbn, axis=2, keepdims=True)
    pad = jnp.zeros((N_CB, K, CDA - CD - 1), jnp.float32)
    cbn_ref[...] = jnp.concatenate([cbn, ch, pad], axis=2)


def _rvq_kernel(z_ref, w_in_ref, in_b_ref, w_out_ref, out_b_ref, cbn_ref,
                cbm_ref, zq_out_ref, codes_ref, lat_ref, loss_ref):
    res = z_ref[0]
    acc = jnp.zeros((D, TT), jnp.float32)
    loss = jnp.float32(0.0)
    ones_row = jnp.ones((1, TT), jnp.float32)
    zeros_pad = jnp.zeros((CDA - CD - 1, TT), jnp.float32)
    for i in range(N_CB):
        z_e = _dot(w_in_ref[i], res, ((1,), (0,)), _PREC) + in_b_ref[i][:, None]

        n = jnp.sqrt(jnp.sum(z_e * z_e, axis=0, keepdims=True))
        enc_n = z_e / jnp.maximum(n, 1e-12)

        enc_aug = jnp.concatenate([enc_n, ones_row, zeros_pad], axis=0)
        s = _dot(cbn_ref[i], enc_aug, ((1,), (0,)), _PREC)
        sg = s.reshape(KQ, 128, TT)
        gm = jnp.max(sg, axis=1)
        qidx = jnp.argmax(gm, axis=0).astype(jnp.int32)
        swin = jnp.zeros((128, TT), jnp.float32)
        for g in range(KQ):
            swin = jnp.where(qidx[None, :] == g, sg[g], swin)
        ridx = jnp.argmax(swin, axis=0).astype(jnp.int32)
        idx = qidx * 128 + ridx

        oh_r = (jax.lax.broadcasted_iota(jnp.int32, (128, TT), 0)
                == ridx[None, :]).astype(jnp.float32)
        z1 = _dot(cbm_ref[i], oh_r, ((0,), (0,)), _PREC_GATHER)
        z_q = jnp.zeros((CD, TT), jnp.float32)
        for qq in range(KQ):
            z_q = jnp.where(qidx[None, :] == qq, z1[qq * CD:(qq + 1) * CD], z_q)

        diff = z_e - z_q
        loss = loss + jnp.sum(diff * diff)

        z_q_proj = (_dot(w_out_ref[i], z_q, ((1,), (0,)), _PREC)
                    + out_b_ref[i][:, None])
        acc = acc + z_q_proj
        res = res - z_q_proj

        codes_ref[0, i, :] = idx
        lat_ref[0, i * CD:(i + 1) * CD, :] = z_e

    zq_out_ref[0] = acc
    loss_ref[0, 0] = loss.reshape(1, 1)


@jax.jit
def kernel(z, in_v, in_g, in_b, out_v, out_g, out_b, codebooks):
    cbm = codebooks.reshape(N_CB, KQ, 128, CD).transpose(0, 2, 1, 3).reshape(
        N_CB, 128, KQ * CD)
    prep_shapes = (
        jax.ShapeDtypeStruct((N_CB, CD, D), jnp.float32),
        jax.ShapeDtypeStruct((N_CB, D, CD), jnp.float32),
        jax.ShapeDtypeStruct((N_CB, K, CDA), jnp.float32),
    )
    w_in, w_out, cbn = pl.pallas_call(
        _prep_kernel, out_shape=prep_shapes,
    )(in_v, in_g, out_v, out_g, codebooks)

    full = lambda shape: pl.BlockSpec(shape, lambda b, t: (0,) * len(shape))
    out_shapes = (
        jax.ShapeDtypeStruct((B, D, T), jnp.float32),
        jax.ShapeDtypeStruct((B, CODES_PAD, T), jnp.int32),
        jax.ShapeDtypeStruct((B, N_CB * CD, T), jnp.float32),
        jax.ShapeDtypeStruct((B, GT, 1, 1), jnp.float32),
    )
    zq_out, codes_pad, latents, loss_part = pl.pallas_call(
        _rvq_kernel,
        grid=(B, GT),
        in_specs=[
            pl.BlockSpec((1, D, TT), lambda b, t: (b, 0, t)),
            full((N_CB, CD, D)),
            full((N_CB, CD)),
            full((N_CB, D, CD)),
            full((N_CB, D)),
            full((N_CB, K, CDA)),
            full((N_CB, 128, KQ * CD)),
        ],
        out_specs=(
            pl.BlockSpec((1, D, TT), lambda b, t: (b, 0, t)),
            pl.BlockSpec((1, CODES_PAD, TT), lambda b, t: (b, 0, t)),
            pl.BlockSpec((1, N_CB * CD, TT), lambda b, t: (b, 0, t)),
            pl.BlockSpec((1, 1, 1, 1), lambda b, t: (b, t, 0, 0)),
        ),
        out_shape=out_shapes,
        compiler_params=pltpu.CompilerParams(
            dimension_semantics=("parallel", "parallel")),
    )(z, w_in, in_b, w_out, out_b, cbn, cbm)
    codes = codes_pad[:, :N_CB, :]
    commit = jnp.sum(loss_part) / jnp.float32(B * CD * T)
    return (zq_out, codes, latents, commit, commit)


# speedup vs baseline: 1.1934x; 1.1934x over previous
import jax
import jax.numpy as jnp
from jax.experimental import pallas as pl
from jax.experimental.pallas import tpu as pltpu

B, D, T = 16, 512, 2048
N_CB, K, CD = 9, 1024, 8

TT = 2048
GT = T // TT
CODES_PAD = 16
KQ = K // 128
CDA = 2 * CD

_PREC = None
_PREC_GATHER = jax.lax.Precision.HIGHEST


def _dot(a, b, dims, precision):
    return jax.lax.dot_general(a, b, (dims, ((), ())),
                               preferred_element_type=jnp.float32,
                               precision=precision)


def _prep_kernel(in_v_ref, in_g_ref, out_v_ref, out_g_ref, cb_ref,
                 w_in_ref, w_out_ref, cbn_ref):
    iv = in_v_ref[...]
    w_in_ref[...] = in_g_ref[...][:, :, None] * iv / jnp.sqrt(
        jnp.sum(iv * iv, axis=2, keepdims=True))
    ov = out_v_ref[...]
    w_out_ref[...] = out_g_ref[...][:, :, None] * ov / jnp.sqrt(
        jnp.sum(ov * ov, axis=2, keepdims=True))
    cb = cb_ref[...]
    cbn = cb / jnp.maximum(
        jnp.sqrt(jnp.sum(cb * cb, axis=2, keepdims=True)), 1e-12)
    ch = -0.5 * jnp.sum(cbn * cbn, axis=2, keepdims=True)
    pad = jnp.zeros((N_CB, K, CDA - CD - 1), jnp.float32)
    cbn_ref[...] = jnp.concatenate([cbn, ch, pad], axis=2)


def _rvq_kernel(z_ref, w_in_ref, in_b_ref, w_out_ref, out_b_ref, cbn_ref,
                cbm_ref, zq_out_ref, codes_ref, lat_ref, loss_ref):
    res = z_ref[0]
    acc = jnp.zeros((D, TT), jnp.float32)
    loss = jnp.float32(0.0)
    ones_row = jnp.ones((1, TT), jnp.float32)
    zeros_pad = jnp.zeros((CDA - CD - 1, TT), jnp.float32)
    for i in range(N_CB):
        z_e = _dot(w_in_ref[i], res, ((1,), (0,)), _PREC) + in_b_ref[i][:, None]

        n = jnp.sqrt(jnp.sum(z_e * z_e, axis=0, keepdims=True))
        enc_n = z_e / jnp.maximum(n, 1e-12)

        enc_aug = jnp.concatenate([enc_n, ones_row, zeros_pad], axis=0)
        s = _dot(cbn_ref[i], enc_aug, ((1,), (0,)), _PREC)
        idx = jnp.argmax(s, axis=0).astype(jnp.int32)
        ridx = idx & 127
        qidx = idx >> 7

        oh_r = (jax.lax.broadcasted_iota(jnp.int32, (128, TT), 0)
                == ridx[None, :]).astype(jnp.float32)
        z1 = _dot(cbm_ref[i], oh_r, ((0,), (0,)), _PREC_GATHER)
        z_q = jnp.zeros((CD, TT), jnp.float32)
        for qq in range(KQ):
            z_q = jnp.where(qidx[None, :] == qq, z1[qq * CD:(qq + 1) * CD], z_q)

        diff = z_e - z_q
        loss = loss + jnp.sum(diff * diff)

        z_q_proj = (_dot(w_out_ref[i], z_q, ((1,), (0,)), _PREC)
                    + out_b_ref[i][:, None])
        acc = acc + z_q_proj
        res = res - z_q_proj

        codes_ref[0, i, :] = idx
        lat_ref[0, i * CD:(i + 1) * CD, :] = z_e

    zq_out_ref[0] = acc
    loss_ref[0, 0] = loss.reshape(1, 1)


@jax.jit
def kernel(z, in_v, in_g, in_b, out_v, out_g, out_b, codebooks):
    cbm = codebooks.reshape(N_CB, KQ, 128, CD).transpose(0, 2, 1, 3).reshape(
        N_CB, 128, KQ * CD)
    prep_shapes = (
        jax.ShapeDtypeStruct((N_CB, CD, D), jnp.float32),
        jax.ShapeDtypeStruct((N_CB, D, CD), jnp.float32),
        jax.ShapeDtypeStruct((N_CB, K, CDA), jnp.float32),
    )
    w_in, w_out, cbn = pl.pallas_call(
        _prep_kernel, out_shape=prep_shapes,
    )(in_v, in_g, out_v, out_g, codebooks)

    full = lambda shape: pl.BlockSpec(shape, lambda b, t: (0,) * len(shape))
    out_shapes = (
        jax.ShapeDtypeStruct((B, D, T), jnp.float32),
        jax.ShapeDtypeStruct((B, CODES_PAD, T), jnp.int32),
        jax.ShapeDtypeStruct((B, N_CB * CD, T), jnp.float32),
        jax.ShapeDtypeStruct((B, GT, 1, 1), jnp.float32),
    )
    zq_out, codes_pad, latents, loss_part = pl.pallas_call(
        _rvq_kernel,
        grid=(B, GT),
        in_specs=[
            pl.BlockSpec((1, D, TT), lambda b, t: (b, 0, t)),
            full((N_CB, CD, D)),
            full((N_CB, CD)),
            full((N_CB, D, CD)),
            full((N_CB, D)),
            full((N_CB, K, CDA)),
            full((N_CB, 128, KQ * CD)),
        ],
        out_specs=(
            pl.BlockSpec((1, D, TT), lambda b, t: (b, 0, t)),
            pl.BlockSpec((1, CODES_PAD, TT), lambda b, t: (b, 0, t)),
            pl.BlockSpec((1, N_CB * CD, TT), lambda b, t: (b, 0, t)),
            pl.BlockSpec((1, 1, 1, 1), lambda b, t: (b, t, 0, 0)),
        ),
        out_shape=out_shapes,
        compiler_params=pltpu.CompilerParams(
            dimension_semantics=("parallel", "parallel")),
    )(z, w_in, in_b, w_out, out_b, cbn, cbm)
    codes = codes_pad[:, :N_CB, :]
    commit = jnp.sum(loss_part) / jnp.float32(B * CD * T)
    return (zq_out, codes, latents, commit, commit)
